# baseline (speedup 1.0000x reference)
import functools

import jax
import jax.numpy as jnp
from jax import lax
from jax.experimental import pallas as pl
from jax.experimental.pallas import tpu as pltpu
from jax.experimental.pallas import tpu_sc as plsc

N_ROWS = 64
N_COLS = 32768
L = 16
CHUNKS = N_COLS // L
CCHUNKS = CHUNKS // L
NUM_CORES = 2
NUM_SUBCORES = 16
NW = NUM_CORES * NUM_SUBCORES
ROWS_PER_W = N_ROWS // NW
N_ITERS = 24

_mesh = plsc.VectorSubcoreMesh(
    core_axis_name="c", subcore_axis_name="s",
    num_cores=NUM_CORES, num_subcores=NUM_SUBCORES,
)


@functools.partial(
    pl.kernel,
    out_type=jax.ShapeDtypeStruct((N_ROWS, N_COLS), jnp.float32),
    mesh=_mesh,
    scratch_types=[
        pltpu.VMEM((ROWS_PER_W, N_COLS), jnp.float32),
        pltpu.VMEM((CHUNKS + L,), jnp.float32),
        pltpu.VMEM((CHUNKS // 16 + L,), jnp.int32),
        pltpu.VMEM((N_COLS + L,), jnp.float32),
        pltpu.SemaphoreType.DMA,
        pltpu.SemaphoreType.DMA,
        pltpu.SemaphoreType.DMA,
    ],
    compiler_params=pltpu.CompilerParams(needs_layout_passes=False),
)
def _sparsemax_sc(x_hbm, out_hbm, buf, cnts, nzc, cand, sin0, sin1, sout):
    wid = lax.axis_index("c") * NUM_SUBCORES + lax.axis_index("s")
    base = wid * ROWS_PER_W
    cp_in = [
        pltpu.async_copy(x_hbm.at[pl.ds(base + r, 1)],
                         buf.at[pl.ds(r, 1)], sem)
        for r, sem in ((0, sin0), (1, sin1))
    ]

    lane0 = lax.broadcasted_iota(jnp.int32, (L,), 0) == 0

    U = 8

    def _process(r):
        def max_body(i, accs):
            return tuple(
                jnp.maximum(accs[j], buf[r, pl.ds((i * U + j) * L, L)])
                for j in range(U))

        accs = lax.fori_loop(
            0, CHUNKS // U, max_body,
            tuple(jnp.full((L,), -3e38, jnp.float32) for _ in range(U)))
        rm = accs[0]
        for j in range(1, U):
            rm = jnp.maximum(rm, accs[j])
        m = jnp.broadcast_to(jnp.max(rm), (L,))
        lo0 = m - 1.0

        GRP = 16

        def flags_body(g, carry):
            accs = [jnp.zeros((L,), jnp.float32) for _ in range(4)]
            for j in range(0, GRP, 4):
                for q in range(4):
                    v = buf[r, pl.ds((g * GRP + j + q) * L, L)]
                    accs[q] = jnp.where(v > lo0, 1.0, accs[q])
            cnts[pl.ds(g * L, L)] = (accs[0] + accs[1]) + (accs[2] + accs[3])
            return carry

        NGRP = CHUNKS // GRP
        lax.fori_loop(0, NGRP, flags_body, 0)

        NZB = 8

        def nz_body(b, off):
            fvs = [cnts[pl.ds((b * NZB + q) * L, L)] for q in range(NZB)]
            msks = [fv > 0.0 for fv in fvs]
            pcs = [plsc.all_reduce_population_count(mk) for mk in msks]
            for q in range(NZB):
                anyb = pcs[q] > 0
                gid = jnp.broadcast_to(b * NZB + q, (L,)).astype(jnp.int32)
                plsc.store_compressed(nzc.at[pl.ds(off, L)], gid,
                                      mask=anyb & lane0)
                off = off + jnp.minimum(pcs[q][0], 1)
            return off

        nnz = lax.fori_loop(0, NGRP // NZB, nz_body, jnp.int32(0))

        def gather_body(k, off):
            gid = nzc[pl.ds(k, L)][0]
            for h in range(2):
                vs = [buf[r, pl.ds((gid * GRP + h * 8 + j) * L, L)]
                      for j in range(8)]
                msks = [v > lo0 for v in vs]
                pcs = [plsc.all_reduce_population_count(mk) for mk in msks]
                for j in range(8):
                    plsc.store_compressed(cand.at[pl.ds(off, L)], vs[j],
                                          mask=msks[j])
                    off = off + pcs[j][0]
            return off

        off = lax.fori_loop(0, nnz, gather_body, jnp.int32(0))
        plsc.store_compressed(cand.at[pl.ds(off, L)], lo0,
                              mask=jnp.ones((L,), jnp.bool_))
        nch = off // L + 1

        def cs_pass(t):
            def body(j, carry):
                s_acc, c_acc = carry
                v = cand[pl.ds(j * L, L)]
                msk = v > t
                return (s_acc + jnp.where(msk, v, 0.0),
                        c_acc + jnp.where(msk, 1.0, 0.0))

            z = jnp.zeros((L,), jnp.float32)
            s_acc, c_acc = lax.fori_loop(0, nch, body, (z, z))
            return (jnp.broadcast_to(jnp.sum(s_acc), (L,)),
                    jnp.broadcast_to(jnp.sum(c_acc), (L,)))

        def bis_body(_, carry):
            lo, hi = carry
            t = 0.5 * (lo + hi)
            s, c = cs_pass(t)
            f = s - t * c - 1.0
            tnew = (s - 1.0) / jnp.maximum(c, 1.0)
            hi = jnp.where(f > 0.0, hi, t)
            lo = jnp.minimum(jnp.maximum(lo, tnew), hi)
            return lo, hi

        lo, hi = lax.fori_loop(0, N_ITERS, bis_body, (lo0, m))

        s, c = cs_pass(lo)
        tau = jnp.where(c > 0.0, (s - 1.0) / jnp.maximum(c, 1.0), lo)

        def out_body(i, carry):
            v = buf[r, pl.ds(i * L, L)]
            buf[r, pl.ds(i * L, L)] = jnp.maximum(v - carry, 0.0)
            return carry

        lax.fori_loop(0, CHUNKS, out_body, tau, unroll=16)

    cp_in[0].wait()
    _process(0)
    cp_out0 = pltpu.async_copy(buf.at[pl.ds(0, 1)],
                               out_hbm.at[pl.ds(base, 1)], sout)
    cp_in[1].wait()
    _process(1)
    pltpu.sync_copy(buf.at[pl.ds(1, 1)], out_hbm.at[pl.ds(base + 1, 1)])
    cp_out0.wait()


def kernel(input):
    return _sparsemax_sc(input)


# speedup vs baseline: 1.0131x; 1.0131x over previous
import functools

import jax
import jax.numpy as jnp
from jax import lax
from jax.experimental import pallas as pl
from jax.experimental.pallas import tpu as pltpu
from jax.experimental.pallas import tpu_sc as plsc

N_ROWS = 64
N_COLS = 32768
L = 16
CHUNKS = N_COLS // L
CCHUNKS = CHUNKS // L
NUM_CORES = 2
NUM_SUBCORES = 16
NW = NUM_CORES * NUM_SUBCORES
ROWS_PER_W = N_ROWS // NW
N_ITERS = 24

_mesh = plsc.VectorSubcoreMesh(
    core_axis_name="c", subcore_axis_name="s",
    num_cores=NUM_CORES, num_subcores=NUM_SUBCORES,
)


@functools.partial(
    pl.kernel,
    out_type=jax.ShapeDtypeStruct((N_ROWS, N_COLS), jnp.float32),
    mesh=_mesh,
    scratch_types=[
        pltpu.VMEM((ROWS_PER_W, N_COLS), jnp.float32),
        pltpu.VMEM((CHUNKS + L,), jnp.float32),
        pltpu.VMEM((CHUNKS // 16 + L,), jnp.int32),
        pltpu.VMEM((N_COLS + L,), jnp.float32),
        pltpu.SemaphoreType.DMA,
        pltpu.SemaphoreType.DMA,
        pltpu.SemaphoreType.DMA,
    ],
    compiler_params=pltpu.CompilerParams(needs_layout_passes=False),
)
def _sparsemax_sc(x_hbm, out_hbm, buf, cnts, nzc, cand, sin0, sin1, sout):
    wid = lax.axis_index("c") * NUM_SUBCORES + lax.axis_index("s")
    base = wid * ROWS_PER_W
    cp_in = [
        pltpu.async_copy(x_hbm.at[pl.ds(base + r, 1)],
                         buf.at[pl.ds(r, 1)], sem)
        for r, sem in ((0, sin0), (1, sin1))
    ]

    lane0 = lax.broadcasted_iota(jnp.int32, (L,), 0) == 0

    U = 8

    def _process(r):
        def max_body(i, accs):
            return tuple(
                jnp.maximum(accs[j], buf[r, pl.ds((i * U + j) * L, L)])
                for j in range(U))

        accs = lax.fori_loop(
            0, CHUNKS // U, max_body,
            tuple(jnp.full((L,), -3e38, jnp.float32) for _ in range(U)))
        rm = accs[0]
        for j in range(1, U):
            rm = jnp.maximum(rm, accs[j])
        m = jnp.broadcast_to(jnp.max(rm), (L,))
        lo0 = m - 1.0

        GRP = 16

        def flags_body(g, carry):
            acc0 = jnp.zeros((L,), jnp.float32)
            acc1 = jnp.zeros((L,), jnp.float32)
            for j in range(0, GRP, 2):
                v0 = buf[r, pl.ds((g * GRP + j) * L, L)]
                v1 = buf[r, pl.ds((g * GRP + j + 1) * L, L)]
                acc0 = jnp.where(v0 > lo0, 1.0, acc0)
                acc1 = jnp.where(v1 > lo0, 1.0, acc1)
            cnts[pl.ds(g * L, L)] = acc0 + acc1
            return carry

        NGRP = CHUNKS // GRP
        lax.fori_loop(0, NGRP, flags_body, 0)

        NZB = 8

        def nz_body(b, off):
            fvs = [cnts[pl.ds((b * NZB + q) * L, L)] for q in range(NZB)]
            msks = [fv > 0.0 for fv in fvs]
            pcs = [plsc.all_reduce_population_count(mk) for mk in msks]
            for q in range(NZB):
                anyb = pcs[q] > 0
                gid = jnp.broadcast_to(b * NZB + q, (L,)).astype(jnp.int32)
                plsc.store_compressed(nzc.at[pl.ds(off, L)], gid,
                                      mask=anyb & lane0)
                off = off + jnp.minimum(pcs[q][0], 1)
            return off

        nnz = lax.fori_loop(0, NGRP // NZB, nz_body, jnp.int32(0))

        def gather_body(k, off):
            gid = nzc[pl.ds(k, L)][0]
            for h in range(2):
                vs = [buf[r, pl.ds((gid * GRP + h * 8 + j) * L, L)]
                      for j in range(8)]
                msks = [v > lo0 for v in vs]
                pcs = [plsc.all_reduce_population_count(mk) for mk in msks]
                for j in range(8):
                    plsc.store_compressed(cand.at[pl.ds(off, L)], vs[j],
                                          mask=msks[j])
                    off = off + pcs[j][0]
            return off

        off = lax.fori_loop(0, nnz, gather_body, jnp.int32(0))
        plsc.store_compressed(cand.at[pl.ds(off, L)], lo0,
                              mask=jnp.ones((L,), jnp.bool_))
        nch = off // L + 1

        def cs_pass(t):
            def body(j, carry):
                s_acc, c_acc = carry
                v = cand[pl.ds(j * L, L)]
                msk = v > t
                return (s_acc + jnp.where(msk, v, 0.0),
                        c_acc + jnp.where(msk, 1.0, 0.0))

            z = jnp.zeros((L,), jnp.float32)
            s_acc, c_acc = lax.fori_loop(0, nch, body, (z, z))
            return (jnp.broadcast_to(jnp.sum(s_acc), (L,)),
                    jnp.broadcast_to(jnp.sum(c_acc), (L,)))

        def bis_body(_, carry):
            lo, hi = carry
            t = 0.5 * (lo + hi)
            s, c = cs_pass(t)
            f = s - t * c - 1.0
            tnew = (s - 1.0) / jnp.maximum(c, 1.0)
            hi = jnp.where(f > 0.0, hi, t)
            lo = jnp.minimum(jnp.maximum(lo, tnew), hi)
            return lo, hi

        lo, hi = lax.fori_loop(0, N_ITERS, bis_body, (lo0, m))

        s, c = cs_pass(lo)
        tau = jnp.where(c > 0.0, (s - 1.0) / jnp.maximum(c, 1.0), lo)

        def out_body(i, carry):
            v = buf[r, pl.ds(i * L, L)]
            buf[r, pl.ds(i * L, L)] = jnp.maximum(v - carry, 0.0)
            return carry

        lax.fori_loop(0, CHUNKS, out_body, tau, unroll=16)

    cp_in[0].wait()
    _process(0)
    cp_out0 = pltpu.async_copy(buf.at[pl.ds(0, 1)],
                               out_hbm.at[pl.ds(base, 1)], sout)
    cp_in[1].wait()
    _process(1)
    pltpu.sync_copy(buf.at[pl.ds(1, 1)], out_hbm.at[pl.ds(base + 1, 1)])
    cp_out0.wait()


def kernel(input):
    return _sparsemax_sc(input)
